# baseline (speedup 1.0000x reference)
import functools

import jax
import jax.numpy as jnp
from jax import lax
from jax.experimental import pallas as pl
from jax.experimental.pallas import tpu as pltpu
from jax.experimental.pallas import tpu_sc as plsc

_N = 10000
_E = 320000
_D = 128
_G = 64
_NG = 256
_CH = 125
_NW = 32
_EPW = _E // _NW
_CPW = _EPW // _CH
_RPS = 640
_NPAD = 16 * _RPS
_ZR = 320
_DSEG = 640
_NDEG = 16 * _DSEG
_RB = 1000
_NB = _N // _RB


def _make_deg_kernel():
    mesh = plsc.VectorSubcoreMesh(core_axis_name="c", subcore_axis_name="s")

    @functools.partial(
        pl.kernel,
        mesh=mesh,
        out_type=[jax.ShapeDtypeStruct((_NDEG,), jnp.float32),
                  jax.ShapeDtypeStruct((_NDEG,), jnp.float32)],
        compiler_params=pltpu.CompilerParams(use_tc_tiling_on_sc=False),
        scratch_types=[
            pltpu.VMEM((_CPW, _CH), jnp.int32),
            pltpu.VMEM((128,), jnp.float32),
            pltpu.VMEM((_DSEG,), jnp.float32),
            pltpu.VMEM_SHARED((_NDEG,), jnp.float32),
        ],
    )
    def deg_kernel(ei_hbm, out0_hbm, out1_hbm, didx, ones, zbuf, acc):
        c = lax.axis_index("c")
        s = lax.axis_index("s")
        w = c * 16 + s

        def fill(i, _):
            zbuf[pl.ds(i * 16, 16)] = jnp.zeros((16,), jnp.float32)
            return 0

        lax.fori_loop(0, _DSEG // 16, fill, 0)

        def fill1(i, _):
            ones[pl.ds(i * 16, 16)] = jnp.ones((16,), jnp.float32)
            return 0

        lax.fori_loop(0, 8, fill1, 0)
        pltpu.sync_copy(zbuf, acc.at[pl.ds(s * _DSEG, _DSEG)])
        pltpu.sync_copy(ei_hbm.at[1, w], didx)
        plsc.subcore_barrier()

        def body(j, _):
            pltpu.sync_copy(ones.at[pl.ds(0, _CH)], acc.at[didx.at[j]],
                            add=True)
            return 0

        lax.fori_loop(0, _CPW, body, 0)
        plsc.subcore_barrier()

        @pl.when(c == 0)
        def _():
            pltpu.sync_copy(acc.at[pl.ds(s * _DSEG, _DSEG)],
                            out0_hbm.at[pl.ds(s * _DSEG, _DSEG)])

        @pl.when(c == 1)
        def _():
            pltpu.sync_copy(acc.at[pl.ds(s * _DSEG, _DSEG)],
                            out1_hbm.at[pl.ds(s * _DSEG, _DSEG)])

    return deg_kernel


def _make_conv_kernel():
    mesh = plsc.VectorSubcoreMesh(core_axis_name="c", subcore_axis_name="s")

    @functools.partial(
        pl.kernel,
        mesh=mesh,
        out_type=jax.ShapeDtypeStruct((2, _NPAD, _G), jnp.float32),
        compiler_params=pltpu.CompilerParams(use_tc_tiling_on_sc=False),
        scratch_types=[
            pltpu.VMEM((_CPW, _CH), jnp.int32),
            pltpu.VMEM((_CPW, _CH), jnp.int32),
            pltpu.VMEM((_CH, _G), jnp.float32),
            pltpu.VMEM((_CH, _G), jnp.float32),
            pltpu.VMEM((_ZR, _G), jnp.float32),
            pltpu.VMEM_SHARED((_NPAD, _G), jnp.float32),
            pltpu.SemaphoreType.DMA,
            pltpu.SemaphoreType.DMA,
        ],
    )
    def conv_kernel(t_hbm, ei_hbm, out_hbm,
                    sidx, didx, r_a, r_b, zbuf, acc, sem_a, sem_b):
        c = lax.axis_index("c")
        s = lax.axis_index("s")
        w = c * 16 + s

        def fill(i, _):
            zbuf[i, pl.ds(0, 16)] = jnp.zeros((16,), jnp.float32)
            zbuf[i, pl.ds(16, 16)] = jnp.zeros((16,), jnp.float32)
            zbuf[i, pl.ds(32, 16)] = jnp.zeros((16,), jnp.float32)
            zbuf[i, pl.ds(48, 16)] = jnp.zeros((16,), jnp.float32)
            return 0

        lax.fori_loop(0, _ZR, fill, 0)
        pltpu.sync_copy(zbuf, acc.at[pl.ds(s * _RPS, _ZR)])
        pltpu.sync_copy(zbuf, acc.at[pl.ds(s * _RPS + _ZR, _ZR)])
        pltpu.sync_copy(ei_hbm.at[0, w], sidx)
        pltpu.sync_copy(ei_hbm.at[1, w], didx)
        plsc.subcore_barrier()

        def gstart(j, buf, sem):
            pltpu.async_copy(t_hbm.at[sidx.at[j]], buf, sem)

        def gwait(j, buf, sem):
            pltpu.make_async_copy(t_hbm.at[sidx.at[j]], buf, sem).wait()

        gstart(0, r_a, sem_a)

        def body(i, _):
            ja = 2 * i
            jb = ja + 1
            gstart(jb, r_b, sem_b)
            gwait(ja, r_a, sem_a)
            pltpu.sync_copy(r_a, acc.at[didx.at[ja]], add=True)

            @pl.when(jb + 1 < _CPW)
            def _():
                gstart(jb + 1, r_a, sem_a)

            gwait(jb, r_b, sem_b)
            pltpu.sync_copy(r_b, acc.at[didx.at[jb]], add=True)
            return 0

        lax.fori_loop(0, _CPW // 2, body, 0)
        plsc.subcore_barrier()
        pltpu.sync_copy(acc.at[pl.ds(s * _RPS, _RPS)],
                        out_hbm.at[c, pl.ds(s * _RPS, _RPS)])

    return conv_kernel


_deg_call = _make_deg_kernel()
_conv_call = _make_conv_kernel()


def _dot16(a, b):
    return jnp.dot(a.astype(jnp.bfloat16), b.astype(jnp.bfloat16),
                   preferred_element_type=jnp.float32)

def _tc0_body(x_ref, w_ref, d0_ref, d1_ref, t_ref, dinv_ref):
    deg = d0_ref[...] + d1_ref[...] + 1.0
    dinv = lax.rsqrt(deg)
    t = _dot16(x_ref[...], w_ref[...])
    t_ref[...] = t * dinv
    dinv_ref[...] = dinv


_tc0_call = pl.pallas_call(
    _tc0_body,
    grid=(_NB,),
    in_specs=[
        pl.BlockSpec((_RB, _D), lambda i: (i, 0)),
        pl.BlockSpec((_D, _G), lambda i: (0, 0)),
        pl.BlockSpec((_RB, 1), lambda i: (i, 0)),
        pl.BlockSpec((_RB, 1), lambda i: (i, 0)),
    ],
    out_specs=[
        pl.BlockSpec((_RB, _G), lambda i: (i, 0)),
        pl.BlockSpec((_RB, 1), lambda i: (i, 0)),
    ],
    out_shape=[
        jax.ShapeDtypeStruct((_N, _G), jnp.float32),
        jax.ShapeDtypeStruct((_N, 1), jnp.float32),
    ],
)


def _make_tc_conv(relu):
    def body(s_ref, t_ref, dinv_ref, b_ref, g_ref, bb_ref, w_ref,
             t_next_ref, stats_ref):
        i = pl.program_id(0)
        u = (s_ref[0] + s_ref[1] + t_ref[...]) * dinv_ref[...] + b_ref[...]
        if relu:
            u = jnp.maximum(u, 0.0)

        @pl.when(i == 0)
        def _():
            stats_ref[2, :] = jnp.sum(u, axis=0) * (1.0 / _RB)

        a = stats_ref[2, :]
        d = u - a
        ps = jnp.sum(d, axis=0)
        pss = jnp.sum(d * d, axis=0)

        @pl.when(i == 0)
        def _():
            stats_ref[0, :] = ps
            stats_ref[1, :] = pss

        @pl.when((i > 0) & (i < _NB))
        def _():
            stats_ref[0, :] += ps
            stats_ref[1, :] += pss

        @pl.when(i >= _NB)
        def _():
            dm = stats_ref[0, :] * (1.0 / _N)
            mu = a + dm
            var = stats_ref[1, :] * (1.0 / _N) - dm * dm
            sc = lax.rsqrt(var + 1e-5) * g_ref[...]
            h = (u - mu) * sc + bb_ref[...]
            t_next_ref[...] = _dot16(h, w_ref[...]) * dinv_ref[...]

    return pl.pallas_call(
        body,
        grid=(2 * _NB,),
        in_specs=[
            pl.BlockSpec((2, _RB, _G), lambda i: (0, lax.rem(i, _NB), 0)),
            pl.BlockSpec((_RB, _G), lambda i: (lax.rem(i, _NB), 0)),
            pl.BlockSpec((_RB, 1), lambda i: (lax.rem(i, _NB), 0)),
            pl.BlockSpec((_G,), lambda i: (0,)),
            pl.BlockSpec((_G,), lambda i: (0,)),
            pl.BlockSpec((_G,), lambda i: (0,)),
            pl.BlockSpec((_G, _G), lambda i: (0, 0)),
        ],
        out_specs=pl.BlockSpec((_RB, _G), lambda i: (lax.max(i - _NB, 0), 0)),
        out_shape=jax.ShapeDtypeStruct((_N, _G), jnp.float32),
        scratch_shapes=[pltpu.VMEM((3, _G), jnp.float32)],
    )


_tc_conv_relu = _make_tc_conv(True)
_tc_conv_plain = _make_tc_conv(False)


def _bn_val(x, g, b):
    mu = jnp.mean(x, axis=0)
    d = x - mu
    var = jnp.mean(d * d, axis=0)
    return d * lax.rsqrt(var + 1e-5) * g + b


def _tc_tail_body(s_ref, t_ref, dinv_ref, b_ref, g_ref, bb_ref, batch_ref,
                  wm_ref, bm_ref, g2_ref, b2_ref, wd_ref, bd_ref,
                  gd_ref, bdn_ref, wo_ref, bo_ref,
                  out_ref, stats_ref, pooled_ref):
    i = pl.program_id(0)
    u = (s_ref[0] + s_ref[1] + t_ref[...]) * dinv_ref[...] + b_ref[...]

    @pl.when(i == 0)
    def _():
        stats_ref[2, :] = jnp.sum(u, axis=0) * (1.0 / _RB)

    a = stats_ref[2, :]
    d = u - a
    ps = jnp.sum(d, axis=0)
    pss = jnp.sum(d * d, axis=0)

    @pl.when(i == 0)
    def _():
        stats_ref[0, :] = ps
        stats_ref[1, :] = pss

    @pl.when((i > 0) & (i < _NB))
    def _():
        stats_ref[0, :] += ps
        stats_ref[1, :] += pss

    @pl.when((i >= _NB) & (i < 2 * _NB))
    def _():
        dm = stats_ref[0, :] * (1.0 / _N)
        mu = a + dm
        var = stats_ref[1, :] * (1.0 / _N) - dm * dm
        sc = lax.rsqrt(var + 1e-5) * g_ref[...]
        h = (u - mu) * sc + bb_ref[...]
        mask = (batch_ref[...] == lax.broadcasted_iota(jnp.int32, (1, _NG), 1)
                ).astype(jnp.float32)
        pp = lax.dot_general(mask, h, (((0,), (0,)), ((), ())),
                             preferred_element_type=jnp.float32,
                             precision=lax.Precision.HIGHEST)

        @pl.when(i == _NB)
        def _():
            pooled_ref[...] = pp

        @pl.when(i > _NB)
        def _():
            pooled_ref[...] += pp

    @pl.when(i == 2 * _NB)
    def _():
        p = pooled_ref[...]
        hm = _dot16(p, wm_ref[...])
        hm = jnp.maximum(hm + bm_ref[...], 0.0)
        hm = _bn_val(hm, g2_ref[...], b2_ref[...])
        for k in range(3):
            hm = _dot16(hm, wd_ref[k])
            hm = jnp.maximum(hm + bd_ref[k], 0.0)
            hm = _bn_val(hm, gd_ref[k], bdn_ref[k])
        out_ref[...] = _dot16(hm, wo_ref[...]) + bo_ref[...]


_tc_tail_call = pl.pallas_call(
    _tc_tail_body,
    grid=(2 * _NB + 1,),
    in_specs=[
        pl.BlockSpec((2, _RB, _G), lambda i: (0, lax.rem(i, _NB), 0)),
        pl.BlockSpec((_RB, _G), lambda i: (lax.rem(i, _NB), 0)),
        pl.BlockSpec((_RB, 1), lambda i: (lax.rem(i, _NB), 0)),
        pl.BlockSpec((_G,), lambda i: (0,)),
        pl.BlockSpec((_G,), lambda i: (0,)),
        pl.BlockSpec((_G,), lambda i: (0,)),
        pl.BlockSpec((_RB, 1), lambda i: (lax.rem(i, _NB), 0)),
        pl.BlockSpec((_G, _G), lambda i: (0, 0)),
        pl.BlockSpec((_G,), lambda i: (0,)),
        pl.BlockSpec((_G,), lambda i: (0,)),
        pl.BlockSpec((_G,), lambda i: (0,)),
        pl.BlockSpec((3, _G, _G), lambda i: (0, 0, 0)),
        pl.BlockSpec((3, _G), lambda i: (0, 0)),
        pl.BlockSpec((3, _G), lambda i: (0, 0)),
        pl.BlockSpec((3, _G), lambda i: (0, 0)),
        pl.BlockSpec((_G, 1), lambda i: (0, 0)),
        pl.BlockSpec((1,), lambda i: (0,)),
    ],
    out_specs=pl.BlockSpec((_NG, 1), lambda i: (0, 0)),
    out_shape=jax.ShapeDtypeStruct((_NG, 1), jnp.float32),
    scratch_shapes=[pltpu.VMEM((3, _G), jnp.float32),
                    pltpu.VMEM((_NG, _G), jnp.float32)],
)


def kernel(x, edge_index, batch, W1, b1, bn1_g, bn1_b, Wh, bh, bnh_g, bnh_b,
           Wm, bm, bn2_g, bn2_b, Wd, bd, bnd_g, bnd_b, Wo, bo):
    ei4d = edge_index.reshape(2, _NW, _CPW, _CH)
    batch2d = batch.reshape(_N, 1)

    deg0, deg1 = _deg_call(ei4d)
    t, dinv = _tc0_call(x, W1, deg0.reshape(_NDEG, 1), deg1.reshape(_NDEG, 1))

    biases = [b1, bh[0], bh[1], bh[2]]
    gammas = [bn1_g, bnh_g[0], bnh_g[1], bnh_g[2]]
    betas = [bn1_b, bnh_b[0], bnh_b[1], bnh_b[2]]
    nextw = [Wh[0], Wh[1], Wh[2]]

    for k in range(3):
        s_part = _conv_call(t, ei4d)
        tc = _tc_conv_relu if k == 0 else _tc_conv_plain
        t = tc(s_part, t, dinv, biases[k], gammas[k], betas[k], nextw[k])

    s_part = _conv_call(t, ei4d)
    return _tc_tail_call(s_part, t, dinv, biases[3], gammas[3], betas[3],
                         batch2d, Wm, bm, bn2_g, bn2_b, Wd, bd,
                         bnd_g, bnd_b, Wo, bo)


# speedup vs baseline: 1.0471x; 1.0471x over previous
import functools

import jax
import jax.numpy as jnp
from jax import lax
from jax.experimental import pallas as pl
from jax.experimental.pallas import tpu as pltpu
from jax.experimental.pallas import tpu_sc as plsc

_N = 10000
_E = 320000
_D = 128
_G = 64
_NG = 256
_CH = 125
_NW = 32
_EPW = _E // _NW
_CPW = _EPW // _CH
_RPS = 640
_NPAD = 16 * _RPS
_ZR = 320
_DSEG = 640
_NDEG = 16 * _DSEG
_RB = 1000
_NB = _N // _RB


def _make_deg_kernel():
    mesh = plsc.VectorSubcoreMesh(core_axis_name="c", subcore_axis_name="s")

    @functools.partial(
        pl.kernel,
        mesh=mesh,
        out_type=[jax.ShapeDtypeStruct((_NDEG,), jnp.float32),
                  jax.ShapeDtypeStruct((_NDEG,), jnp.float32)],
        compiler_params=pltpu.CompilerParams(use_tc_tiling_on_sc=False),
        scratch_types=[
            pltpu.VMEM((_CPW, _CH), jnp.int32),
            pltpu.VMEM((128,), jnp.float32),
            pltpu.VMEM((_DSEG,), jnp.float32),
            pltpu.VMEM_SHARED((_NDEG,), jnp.float32),
        ],
    )
    def deg_kernel(ei_hbm, out0_hbm, out1_hbm, didx, ones, zbuf, acc):
        c = lax.axis_index("c")
        s = lax.axis_index("s")
        w = c * 16 + s

        def fill(i, _):
            zbuf[pl.ds(i * 16, 16)] = jnp.zeros((16,), jnp.float32)
            return 0

        lax.fori_loop(0, _DSEG // 16, fill, 0)

        def fill1(i, _):
            ones[pl.ds(i * 16, 16)] = jnp.ones((16,), jnp.float32)
            return 0

        lax.fori_loop(0, 8, fill1, 0)
        pltpu.sync_copy(zbuf, acc.at[pl.ds(s * _DSEG, _DSEG)])
        pltpu.sync_copy(ei_hbm.at[1, w], didx)
        plsc.subcore_barrier()

        def body(j, _):
            pltpu.sync_copy(ones.at[pl.ds(0, _CH)], acc.at[didx.at[j]],
                            add=True)
            return 0

        lax.fori_loop(0, _CPW, body, 0)
        plsc.subcore_barrier()

        @pl.when(c == 0)
        def _():
            pltpu.sync_copy(acc.at[pl.ds(s * _DSEG, _DSEG)],
                            out0_hbm.at[pl.ds(s * _DSEG, _DSEG)])

        @pl.when(c == 1)
        def _():
            pltpu.sync_copy(acc.at[pl.ds(s * _DSEG, _DSEG)],
                            out1_hbm.at[pl.ds(s * _DSEG, _DSEG)])

    return deg_kernel


def _make_conv_kernel():
    mesh = plsc.VectorSubcoreMesh(core_axis_name="c", subcore_axis_name="s")

    @functools.partial(
        pl.kernel,
        mesh=mesh,
        out_type=jax.ShapeDtypeStruct((2, _NPAD, _G), jnp.float32),
        compiler_params=pltpu.CompilerParams(use_tc_tiling_on_sc=False),
        scratch_types=[
            pltpu.VMEM((_CPW, _CH), jnp.int32),
            pltpu.VMEM((_CPW, _CH), jnp.int32),
            pltpu.VMEM((_CH, _G), jnp.float32),
            pltpu.VMEM((_CH, _G), jnp.float32),
            pltpu.VMEM((_ZR, _G), jnp.float32),
            pltpu.VMEM_SHARED((_NPAD, _G), jnp.float32),
            pltpu.SemaphoreType.DMA,
            pltpu.SemaphoreType.DMA,
        ],
    )
    def conv_kernel(t_hbm, ei_hbm, out_hbm,
                    sidx, didx, r_a, r_b, zbuf, acc, sem_a, sem_b):
        c = lax.axis_index("c")
        s = lax.axis_index("s")
        w = c * 16 + s

        pltpu.sync_copy(ei_hbm.at[0, w], sidx)
        pltpu.sync_copy(ei_hbm.at[1, w], didx)

        def gstart(j, buf, sem):
            pltpu.async_copy(t_hbm.at[sidx.at[j]], buf, sem)

        def gwait(j, buf, sem):
            pltpu.make_async_copy(t_hbm.at[sidx.at[j]], buf, sem).wait()

        gstart(0, r_a, sem_a)
        gstart(1, r_b, sem_b)

        def fill(i, _):
            zbuf[i, pl.ds(0, 16)] = jnp.zeros((16,), jnp.float32)
            zbuf[i, pl.ds(16, 16)] = jnp.zeros((16,), jnp.float32)
            zbuf[i, pl.ds(32, 16)] = jnp.zeros((16,), jnp.float32)
            zbuf[i, pl.ds(48, 16)] = jnp.zeros((16,), jnp.float32)
            return 0

        lax.fori_loop(0, _ZR, fill, 0)
        pltpu.sync_copy(zbuf, acc.at[pl.ds(s * _RPS, _ZR)])
        pltpu.sync_copy(zbuf, acc.at[pl.ds(s * _RPS + _ZR, _ZR)])
        plsc.subcore_barrier()

        def body(i, _):
            j0 = 4 * i

            def step(off, buf, sem):
                j = j0 + off
                gwait(j, buf, sem)
                pltpu.sync_copy(buf, acc.at[didx.at[j]], add=True)

                @pl.when(j + 2 < _CPW)
                def _():
                    gstart(j + 2, buf, sem)

            step(0, r_a, sem_a)
            step(1, r_b, sem_b)
            step(2, r_a, sem_a)
            step(3, r_b, sem_b)
            return 0

        lax.fori_loop(0, _CPW // 4, body, 0)
        plsc.subcore_barrier()
        pltpu.sync_copy(acc.at[pl.ds(s * _RPS, _RPS)],
                        out_hbm.at[c, pl.ds(s * _RPS, _RPS)])

    return conv_kernel


_deg_call = _make_deg_kernel()
_conv_call = _make_conv_kernel()


def _dot16(a, b):
    return jnp.dot(a.astype(jnp.bfloat16), b.astype(jnp.bfloat16),
                   preferred_element_type=jnp.float32)

def _tc0_body(x_ref, w_ref, d0_ref, d1_ref, t_ref, dinv_ref):
    deg = d0_ref[...] + d1_ref[...] + 1.0
    dinv = lax.rsqrt(deg)
    t = _dot16(x_ref[...], w_ref[...])
    t_ref[...] = t * dinv
    dinv_ref[...] = dinv


_tc0_call = pl.pallas_call(
    _tc0_body,
    grid=(_NB,),
    in_specs=[
        pl.BlockSpec((_RB, _D), lambda i: (i, 0)),
        pl.BlockSpec((_D, _G), lambda i: (0, 0)),
        pl.BlockSpec((_RB, 1), lambda i: (i, 0)),
        pl.BlockSpec((_RB, 1), lambda i: (i, 0)),
    ],
    out_specs=[
        pl.BlockSpec((_RB, _G), lambda i: (i, 0)),
        pl.BlockSpec((_RB, 1), lambda i: (i, 0)),
    ],
    out_shape=[
        jax.ShapeDtypeStruct((_N, _G), jnp.float32),
        jax.ShapeDtypeStruct((_N, 1), jnp.float32),
    ],
)


def _make_tc_conv(relu):
    def body(s_ref, t_ref, dinv_ref, b_ref, g_ref, bb_ref, w_ref,
             t_next_ref, stats_ref, u_scr):
        i = pl.program_id(0)
        j = lax.rem(i, _NB)

        @pl.when(i < _NB)
        def _():
            u = (s_ref[0] + s_ref[1] + t_ref[...]) * dinv_ref[...] + b_ref[...]
            if relu:
                u = jnp.maximum(u, 0.0)
            u_scr[pl.ds(j * _RB, _RB), :] = u

            @pl.when(i == 0)
            def _():
                stats_ref[2, :] = jnp.sum(u, axis=0) * (1.0 / _RB)

            a = stats_ref[2, :]
            d = u - a
            ps = jnp.sum(d, axis=0)
            pss = jnp.sum(d * d, axis=0)

            @pl.when(i == 0)
            def _():
                stats_ref[0, :] = ps
                stats_ref[1, :] = pss

            @pl.when(i > 0)
            def _():
                stats_ref[0, :] += ps
                stats_ref[1, :] += pss

        @pl.when(i >= _NB)
        def _():
            a = stats_ref[2, :]
            dm = stats_ref[0, :] * (1.0 / _N)
            mu = a + dm
            var = stats_ref[1, :] * (1.0 / _N) - dm * dm
            sc = lax.rsqrt(var + 1e-5) * g_ref[...]
            u = u_scr[pl.ds(j * _RB, _RB), :]
            h = (u - mu) * sc + bb_ref[...]
            t_next_ref[...] = _dot16(h, w_ref[...]) * dinv_ref[...]

    return pl.pallas_call(
        body,
        grid=(2 * _NB,),
        in_specs=[
            pl.BlockSpec((2, _RB, _G),
                         lambda i: (0, jnp.where(i < _NB, i, 0), 0)),
            pl.BlockSpec((_RB, _G), lambda i: (jnp.where(i < _NB, i, 0), 0)),
            pl.BlockSpec((_RB, 1), lambda i: (lax.rem(i, _NB), 0)),
            pl.BlockSpec((_G,), lambda i: (0,)),
            pl.BlockSpec((_G,), lambda i: (0,)),
            pl.BlockSpec((_G,), lambda i: (0,)),
            pl.BlockSpec((_G, _G), lambda i: (0, 0)),
        ],
        out_specs=pl.BlockSpec((_RB, _G), lambda i: (lax.max(i - _NB, 0), 0)),
        out_shape=jax.ShapeDtypeStruct((_N, _G), jnp.float32),
        scratch_shapes=[pltpu.VMEM((3, _G), jnp.float32),
                        pltpu.VMEM((_N, _G), jnp.float32)],
    )


_tc_conv_relu = _make_tc_conv(True)
_tc_conv_plain = _make_tc_conv(False)


def _bn_val(x, g, b):
    mu = jnp.mean(x, axis=0)
    d = x - mu
    var = jnp.mean(d * d, axis=0)
    return d * lax.rsqrt(var + 1e-5) * g + b


def _tc_tail_body(s_ref, t_ref, dinv_ref, b_ref, g_ref, bb_ref, batch_ref,
                  wm_ref, bm_ref, g2_ref, b2_ref, wd_ref, bd_ref,
                  gd_ref, bdn_ref, wo_ref, bo_ref,
                  out_ref, stats_ref, pooled_ref, u_scr):
    i = pl.program_id(0)
    j = lax.rem(i, _NB)

    @pl.when(i < _NB)
    def _():
        u = (s_ref[0] + s_ref[1] + t_ref[...]) * dinv_ref[...] + b_ref[...]
        u_scr[pl.ds(j * _RB, _RB), :] = u

        @pl.when(i == 0)
        def _():
            stats_ref[2, :] = jnp.sum(u, axis=0) * (1.0 / _RB)

        a = stats_ref[2, :]
        d = u - a
        ps = jnp.sum(d, axis=0)
        pss = jnp.sum(d * d, axis=0)

        @pl.when(i == 0)
        def _():
            stats_ref[0, :] = ps
            stats_ref[1, :] = pss

        @pl.when(i > 0)
        def _():
            stats_ref[0, :] += ps
            stats_ref[1, :] += pss

    @pl.when((i >= _NB) & (i < 2 * _NB))
    def _():
        a = stats_ref[2, :]
        dm = stats_ref[0, :] * (1.0 / _N)
        mu = a + dm
        var = stats_ref[1, :] * (1.0 / _N) - dm * dm
        sc = lax.rsqrt(var + 1e-5) * g_ref[...]
        u = u_scr[pl.ds(j * _RB, _RB), :]
        h = (u - mu) * sc + bb_ref[...]
        mask = (batch_ref[...] == lax.broadcasted_iota(jnp.int32, (1, _NG), 1)
                ).astype(jnp.float32)
        pp = lax.dot_general(mask, h, (((0,), (0,)), ((), ())),
                             preferred_element_type=jnp.float32,
                             precision=lax.Precision.HIGHEST)

        @pl.when(i == _NB)
        def _():
            pooled_ref[...] = pp

        @pl.when(i > _NB)
        def _():
            pooled_ref[...] += pp

    @pl.when(i == 2 * _NB)
    def _():
        p = pooled_ref[...]
        hm = _dot16(p, wm_ref[...])
        hm = jnp.maximum(hm + bm_ref[...], 0.0)
        hm = _bn_val(hm, g2_ref[...], b2_ref[...])
        for k in range(3):
            hm = _dot16(hm, wd_ref[k])
            hm = jnp.maximum(hm + bd_ref[k], 0.0)
            hm = _bn_val(hm, gd_ref[k], bdn_ref[k])
        out_ref[...] = _dot16(hm, wo_ref[...]) + bo_ref[...]


_tc_tail_call = pl.pallas_call(
    _tc_tail_body,
    grid=(2 * _NB + 1,),
    in_specs=[
        pl.BlockSpec((2, _RB, _G),
                     lambda i: (0, jnp.where(i < _NB, i, 0), 0)),
        pl.BlockSpec((_RB, _G), lambda i: (jnp.where(i < _NB, i, 0), 0)),
        pl.BlockSpec((_RB, 1), lambda i: (lax.rem(i, _NB), 0)),
        pl.BlockSpec((_G,), lambda i: (0,)),
        pl.BlockSpec((_G,), lambda i: (0,)),
        pl.BlockSpec((_G,), lambda i: (0,)),
        pl.BlockSpec((_RB, 1),
                     lambda i: (jnp.where((i >= _NB) & (i < 2 * _NB),
                                          i - _NB, 0), 0)),
        pl.BlockSpec((_G, _G), lambda i: (0, 0)),
        pl.BlockSpec((_G,), lambda i: (0,)),
        pl.BlockSpec((_G,), lambda i: (0,)),
        pl.BlockSpec((_G,), lambda i: (0,)),
        pl.BlockSpec((3, _G, _G), lambda i: (0, 0, 0)),
        pl.BlockSpec((3, _G), lambda i: (0, 0)),
        pl.BlockSpec((3, _G), lambda i: (0, 0)),
        pl.BlockSpec((3, _G), lambda i: (0, 0)),
        pl.BlockSpec((_G, 1), lambda i: (0, 0)),
        pl.BlockSpec((1,), lambda i: (0,)),
    ],
    out_specs=pl.BlockSpec((_NG, 1), lambda i: (0, 0)),
    out_shape=jax.ShapeDtypeStruct((_NG, 1), jnp.float32),
    scratch_shapes=[pltpu.VMEM((3, _G), jnp.float32),
                    pltpu.VMEM((_NG, _G), jnp.float32),
                    pltpu.VMEM((_N, _G), jnp.float32)],
)


def kernel(x, edge_index, batch, W1, b1, bn1_g, bn1_b, Wh, bh, bnh_g, bnh_b,
           Wm, bm, bn2_g, bn2_b, Wd, bd, bnd_g, bnd_b, Wo, bo):
    ei4d = edge_index.reshape(2, _NW, _CPW, _CH)
    batch2d = batch.reshape(_N, 1)

    deg0, deg1 = _deg_call(ei4d)
    t, dinv = _tc0_call(x, W1, deg0.reshape(_NDEG, 1), deg1.reshape(_NDEG, 1))

    biases = [b1, bh[0], bh[1], bh[2]]
    gammas = [bn1_g, bnh_g[0], bnh_g[1], bnh_g[2]]
    betas = [bn1_b, bnh_b[0], bnh_b[1], bnh_b[2]]
    nextw = [Wh[0], Wh[1], Wh[2]]

    for k in range(3):
        s_part = _conv_call(t, ei4d)
        tc = _tc_conv_relu if k == 0 else _tc_conv_plain
        t = tc(s_part, t, dinv, biases[k], gammas[k], betas[k], nextw[k])

    s_part = _conv_call(t, ei4d)
    return _tc_tail_call(s_part, t, dinv, biases[3], gammas[3], betas[3],
                         batch2d, Wm, bm, bn2_g, bn2_b, Wd, bd,
                         bnd_g, bnd_b, Wo, bo)


# speedup vs baseline: 1.1903x; 1.1367x over previous
import functools

import jax
import jax.numpy as jnp
from jax import lax
from jax.experimental import pallas as pl
from jax.experimental.pallas import tpu as pltpu
from jax.experimental.pallas import tpu_sc as plsc

_N = 10000
_E = 320000
_D = 128
_G = 64
_NG = 256
_CH = 125
_NW = 32
_EPW = _E // _NW
_CPW = _EPW // _CH
_RPS = 640
_NPAD = 16 * _RPS
_ZR = 320
_DSEG = 640
_NDEG = 16 * _DSEG
_RB = 1000
_NB = _N // _RB


def _make_deg_kernel():
    mesh = plsc.VectorSubcoreMesh(core_axis_name="c", subcore_axis_name="s")

    @functools.partial(
        pl.kernel,
        mesh=mesh,
        out_type=[jax.ShapeDtypeStruct((_NDEG,), jnp.float32),
                  jax.ShapeDtypeStruct((_NDEG,), jnp.float32)],
        compiler_params=pltpu.CompilerParams(use_tc_tiling_on_sc=False),
        scratch_types=[
            pltpu.VMEM((_CPW, _CH), jnp.int32),
            pltpu.VMEM((128,), jnp.float32),
            pltpu.VMEM((_DSEG,), jnp.float32),
            pltpu.VMEM_SHARED((_NDEG,), jnp.float32),
        ],
    )
    def deg_kernel(ei_hbm, out0_hbm, out1_hbm, didx, ones, zbuf, acc):
        c = lax.axis_index("c")
        s = lax.axis_index("s")
        w = c * 16 + s

        def fill(i, _):
            zbuf[pl.ds(i * 16, 16)] = jnp.zeros((16,), jnp.float32)
            return 0

        lax.fori_loop(0, _DSEG // 16, fill, 0)

        def fill1(i, _):
            ones[pl.ds(i * 16, 16)] = jnp.ones((16,), jnp.float32)
            return 0

        lax.fori_loop(0, 8, fill1, 0)
        pltpu.sync_copy(zbuf, acc.at[pl.ds(s * _DSEG, _DSEG)])
        pltpu.sync_copy(ei_hbm.at[1, w], didx)
        plsc.subcore_barrier()

        def body(j, _):
            pltpu.sync_copy(ones.at[pl.ds(0, _CH)], acc.at[didx.at[j]],
                            add=True)
            return 0

        lax.fori_loop(0, _CPW, body, 0)
        plsc.subcore_barrier()

        @pl.when(c == 0)
        def _():
            pltpu.sync_copy(acc.at[pl.ds(s * _DSEG, _DSEG)],
                            out0_hbm.at[pl.ds(s * _DSEG, _DSEG)])

        @pl.when(c == 1)
        def _():
            pltpu.sync_copy(acc.at[pl.ds(s * _DSEG, _DSEG)],
                            out1_hbm.at[pl.ds(s * _DSEG, _DSEG)])

    return deg_kernel


def _make_conv_kernel():
    mesh = plsc.VectorSubcoreMesh(core_axis_name="c", subcore_axis_name="s")

    @functools.partial(
        pl.kernel,
        mesh=mesh,
        out_type=jax.ShapeDtypeStruct((2, _NPAD, _G), jnp.float32),
        compiler_params=pltpu.CompilerParams(use_tc_tiling_on_sc=False),
        scratch_types=[
            pltpu.VMEM((_CPW, _CH), jnp.int32),
            pltpu.VMEM((_CPW, _CH), jnp.int32),
            pltpu.VMEM((_CH, _G), jnp.float32),
            pltpu.VMEM((_CH, _G), jnp.float32),
            pltpu.VMEM((_CH, _G), jnp.float32),
            pltpu.VMEM((_ZR, _G), jnp.float32),
            pltpu.VMEM_SHARED((_NPAD, _G), jnp.float32),
            pltpu.SemaphoreType.DMA,
            pltpu.SemaphoreType.DMA,
            pltpu.SemaphoreType.DMA,
        ],
    )
    def conv_kernel(t_hbm, ei_hbm, out_hbm,
                    sidx, didx, r_a, r_b, r_c, zbuf, acc, sem_a, sem_b, sem_c):
        c = lax.axis_index("c")
        s = lax.axis_index("s")
        w = c * 16 + s

        pltpu.sync_copy(ei_hbm.at[0, w], sidx)
        pltpu.sync_copy(ei_hbm.at[1, w], didx)

        def gstart(j, buf, sem):
            pltpu.async_copy(t_hbm.at[sidx.at[j]], buf, sem)

        def gwait(j, buf, sem):
            pltpu.make_async_copy(t_hbm.at[sidx.at[j]], buf, sem).wait()

        gstart(0, r_a, sem_a)
        gstart(1, r_b, sem_b)
        gstart(2, r_c, sem_c)

        def fill(i, _):
            zbuf[i, pl.ds(0, 16)] = jnp.zeros((16,), jnp.float32)
            zbuf[i, pl.ds(16, 16)] = jnp.zeros((16,), jnp.float32)
            zbuf[i, pl.ds(32, 16)] = jnp.zeros((16,), jnp.float32)
            zbuf[i, pl.ds(48, 16)] = jnp.zeros((16,), jnp.float32)
            return 0

        lax.fori_loop(0, _ZR, fill, 0)
        pltpu.sync_copy(zbuf, acc.at[pl.ds(s * _RPS, _ZR)])
        pltpu.sync_copy(zbuf, acc.at[pl.ds(s * _RPS + _ZR, _ZR)])
        plsc.subcore_barrier()

        def body(i, _):
            j0 = 3 * i

            def step(off, buf, sem):
                j = j0 + off
                gwait(j, buf, sem)
                pltpu.sync_copy(buf, acc.at[didx.at[j]], add=True)

                @pl.when(j + 3 < _CPW)
                def _():
                    gstart(j + 3, buf, sem)

            step(0, r_a, sem_a)
            step(1, r_b, sem_b)
            step(2, r_c, sem_c)
            return 0

        lax.fori_loop(0, _CPW // 3, body, 0)

        gwait(_CPW - 2, r_a, sem_a)
        pltpu.sync_copy(r_a, acc.at[didx.at[_CPW - 2]], add=True)
        gwait(_CPW - 1, r_b, sem_b)
        pltpu.sync_copy(r_b, acc.at[didx.at[_CPW - 1]], add=True)
        plsc.subcore_barrier()
        pltpu.sync_copy(acc.at[pl.ds(s * _RPS, _RPS)],
                        out_hbm.at[c, pl.ds(s * _RPS, _RPS)])

    return conv_kernel


_deg_call = _make_deg_kernel()
_conv_call = _make_conv_kernel()


def _dot16(a, b):
    return jnp.dot(a.astype(jnp.bfloat16), b.astype(jnp.bfloat16),
                   preferred_element_type=jnp.float32)

def _tc0_body(x_ref, w_ref, d0_ref, d1_ref, t_ref, dinv_ref):
    deg = d0_ref[...] + d1_ref[...] + 1.0
    dinv = lax.rsqrt(deg)
    t = _dot16(x_ref[...], w_ref[...])
    t_ref[...] = t * dinv
    dinv_ref[...] = dinv


_tc0_call = pl.pallas_call(
    _tc0_body,
    grid=(_NB,),
    in_specs=[
        pl.BlockSpec((_RB, _D), lambda i: (i, 0)),
        pl.BlockSpec((_D, _G), lambda i: (0, 0)),
        pl.BlockSpec((_RB, 1), lambda i: (i, 0)),
        pl.BlockSpec((_RB, 1), lambda i: (i, 0)),
    ],
    out_specs=[
        pl.BlockSpec((_RB, _G), lambda i: (i, 0)),
        pl.BlockSpec((_RB, 1), lambda i: (i, 0)),
    ],
    out_shape=[
        jax.ShapeDtypeStruct((_N, _G), jnp.float32),
        jax.ShapeDtypeStruct((_N, 1), jnp.float32),
    ],
)


def _make_tc_conv(relu):
    def body(s_ref, t_ref, dinv_ref, b_ref, g_ref, bb_ref, w_ref,
             t_next_ref, stats_ref, u_scr):
        i = pl.program_id(0)
        j = lax.rem(i, _NB)

        @pl.when(i < _NB)
        def _():
            u = (s_ref[0] + s_ref[1] + t_ref[...]) * dinv_ref[...] + b_ref[...]
            if relu:
                u = jnp.maximum(u, 0.0)
            u_scr[pl.ds(j * _RB, _RB), :] = u

            @pl.when(i == 0)
            def _():
                stats_ref[2, :] = jnp.sum(u, axis=0) * (1.0 / _RB)

            a = stats_ref[2, :]
            d = u - a
            ps = jnp.sum(d, axis=0)
            pss = jnp.sum(d * d, axis=0)

            @pl.when(i == 0)
            def _():
                stats_ref[0, :] = ps
                stats_ref[1, :] = pss

            @pl.when(i > 0)
            def _():
                stats_ref[0, :] += ps
                stats_ref[1, :] += pss

        @pl.when(i >= _NB)
        def _():
            a = stats_ref[2, :]
            dm = stats_ref[0, :] * (1.0 / _N)
            mu = a + dm
            var = stats_ref[1, :] * (1.0 / _N) - dm * dm
            sc = lax.rsqrt(var + 1e-5) * g_ref[...]
            u = u_scr[pl.ds(j * _RB, _RB), :]
            h = (u - mu) * sc + bb_ref[...]
            t_next_ref[...] = _dot16(h, w_ref[...]) * dinv_ref[...]

    return pl.pallas_call(
        body,
        grid=(2 * _NB,),
        in_specs=[
            pl.BlockSpec((2, _RB, _G),
                         lambda i: (0, jnp.where(i < _NB, i, 0), 0)),
            pl.BlockSpec((_RB, _G), lambda i: (jnp.where(i < _NB, i, 0), 0)),
            pl.BlockSpec((_RB, 1), lambda i: (lax.rem(i, _NB), 0)),
            pl.BlockSpec((_G,), lambda i: (0,)),
            pl.BlockSpec((_G,), lambda i: (0,)),
            pl.BlockSpec((_G,), lambda i: (0,)),
            pl.BlockSpec((_G, _G), lambda i: (0, 0)),
        ],
        out_specs=pl.BlockSpec((_RB, _G), lambda i: (lax.max(i - _NB, 0), 0)),
        out_shape=jax.ShapeDtypeStruct((_N, _G), jnp.float32),
        scratch_shapes=[pltpu.VMEM((3, _G), jnp.float32),
                        pltpu.VMEM((_N, _G), jnp.float32)],
    )


_tc_conv_relu = _make_tc_conv(True)
_tc_conv_plain = _make_tc_conv(False)


def _bn_val(x, g, b):
    mu = jnp.mean(x, axis=0)
    d = x - mu
    var = jnp.mean(d * d, axis=0)
    return d * lax.rsqrt(var + 1e-5) * g + b


def _tc_tail_body(s_ref, t_ref, dinv_ref, b_ref, g_ref, bb_ref, batch_ref,
                  wm_ref, bm_ref, g2_ref, b2_ref, wd_ref, bd_ref,
                  gd_ref, bdn_ref, wo_ref, bo_ref,
                  out_ref, stats_ref, pooled_ref, u_scr):
    i = pl.program_id(0)
    j = lax.rem(i, _NB)

    @pl.when(i < _NB)
    def _():
        u = (s_ref[0] + s_ref[1] + t_ref[...]) * dinv_ref[...] + b_ref[...]
        u_scr[pl.ds(j * _RB, _RB), :] = u

        @pl.when(i == 0)
        def _():
            stats_ref[2, :] = jnp.sum(u, axis=0) * (1.0 / _RB)

        a = stats_ref[2, :]
        d = u - a
        ps = jnp.sum(d, axis=0)
        pss = jnp.sum(d * d, axis=0)

        @pl.when(i == 0)
        def _():
            stats_ref[0, :] = ps
            stats_ref[1, :] = pss

        @pl.when(i > 0)
        def _():
            stats_ref[0, :] += ps
            stats_ref[1, :] += pss

    @pl.when((i >= _NB) & (i < 2 * _NB))
    def _():
        a = stats_ref[2, :]
        dm = stats_ref[0, :] * (1.0 / _N)
        mu = a + dm
        var = stats_ref[1, :] * (1.0 / _N) - dm * dm
        sc = lax.rsqrt(var + 1e-5) * g_ref[...]
        u = u_scr[pl.ds(j * _RB, _RB), :]
        h = (u - mu) * sc + bb_ref[...]
        mask = (batch_ref[...] == lax.broadcasted_iota(jnp.int32, (1, _NG), 1)
                ).astype(jnp.float32)
        pp = lax.dot_general(mask, h, (((0,), (0,)), ((), ())),
                             preferred_element_type=jnp.float32,
                             precision=lax.Precision.HIGHEST)

        @pl.when(i == _NB)
        def _():
            pooled_ref[...] = pp

        @pl.when(i > _NB)
        def _():
            pooled_ref[...] += pp

    @pl.when(i == 2 * _NB)
    def _():
        p = pooled_ref[...]
        hm = _dot16(p, wm_ref[...])
        hm = jnp.maximum(hm + bm_ref[...], 0.0)
        hm = _bn_val(hm, g2_ref[...], b2_ref[...])
        for k in range(3):
            hm = _dot16(hm, wd_ref[k])
            hm = jnp.maximum(hm + bd_ref[k], 0.0)
            hm = _bn_val(hm, gd_ref[k], bdn_ref[k])
        out_ref[...] = _dot16(hm, wo_ref[...]) + bo_ref[...]


_tc_tail_call = pl.pallas_call(
    _tc_tail_body,
    grid=(2 * _NB + 1,),
    in_specs=[
        pl.BlockSpec((2, _RB, _G),
                     lambda i: (0, jnp.where(i < _NB, i, 0), 0)),
        pl.BlockSpec((_RB, _G), lambda i: (jnp.where(i < _NB, i, 0), 0)),
        pl.BlockSpec((_RB, 1), lambda i: (lax.rem(i, _NB), 0)),
        pl.BlockSpec((_G,), lambda i: (0,)),
        pl.BlockSpec((_G,), lambda i: (0,)),
        pl.BlockSpec((_G,), lambda i: (0,)),
        pl.BlockSpec((_RB, 1),
                     lambda i: (jnp.where((i >= _NB) & (i < 2 * _NB),
                                          i - _NB, 0), 0)),
        pl.BlockSpec((_G, _G), lambda i: (0, 0)),
        pl.BlockSpec((_G,), lambda i: (0,)),
        pl.BlockSpec((_G,), lambda i: (0,)),
        pl.BlockSpec((_G,), lambda i: (0,)),
        pl.BlockSpec((3, _G, _G), lambda i: (0, 0, 0)),
        pl.BlockSpec((3, _G), lambda i: (0, 0)),
        pl.BlockSpec((3, _G), lambda i: (0, 0)),
        pl.BlockSpec((3, _G), lambda i: (0, 0)),
        pl.BlockSpec((_G, 1), lambda i: (0, 0)),
        pl.BlockSpec((1,), lambda i: (0,)),
    ],
    out_specs=pl.BlockSpec((_NG, 1), lambda i: (0, 0)),
    out_shape=jax.ShapeDtypeStruct((_NG, 1), jnp.float32),
    scratch_shapes=[pltpu.VMEM((3, _G), jnp.float32),
                    pltpu.VMEM((_NG, _G), jnp.float32),
                    pltpu.VMEM((_N, _G), jnp.float32)],
)


def kernel(x, edge_index, batch, W1, b1, bn1_g, bn1_b, Wh, bh, bnh_g, bnh_b,
           Wm, bm, bn2_g, bn2_b, Wd, bd, bnd_g, bnd_b, Wo, bo):
    ei4d = edge_index.reshape(2, _NW, _CPW, _CH)
    batch2d = batch.reshape(_N, 1)

    deg0, deg1 = _deg_call(ei4d)
    t, dinv = _tc0_call(x, W1, deg0.reshape(_NDEG, 1), deg1.reshape(_NDEG, 1))

    biases = [b1, bh[0], bh[1], bh[2]]
    gammas = [bn1_g, bnh_g[0], bnh_g[1], bnh_g[2]]
    betas = [bn1_b, bnh_b[0], bnh_b[1], bnh_b[2]]
    nextw = [Wh[0], Wh[1], Wh[2]]

    for k in range(3):
        s_part = _conv_call(t, ei4d)
        tc = _tc_conv_relu if k == 0 else _tc_conv_plain
        t = tc(s_part, t, dinv, biases[k], gammas[k], betas[k], nextw[k])

    s_part = _conv_call(t, ei4d)
    return _tc_tail_call(s_part, t, dinv, biases[3], gammas[3], betas[3],
                         batch2d, Wm, bm, bn2_g, bn2_b, Wd, bd,
                         bnd_g, bnd_b, Wo, bo)


# speedup vs baseline: 1.2239x; 1.0283x over previous
import functools

import jax
import jax.numpy as jnp
from jax import lax
from jax.experimental import pallas as pl
from jax.experimental.pallas import tpu as pltpu
from jax.experimental.pallas import tpu_sc as plsc

_N = 10000
_E = 320000
_D = 128
_G = 64
_NG = 256
_CH = 125
_NW = 32
_EPW = _E // _NW
_CPW = _EPW // _CH
_RPS = 640
_NPAD = 16 * _RPS
_ZR = 320
_DSEG = 640
_NDEG = 16 * _DSEG
_RB = 1000
_NB = _N // _RB


def _make_deg_kernel():
    mesh = plsc.VectorSubcoreMesh(core_axis_name="c", subcore_axis_name="s")

    @functools.partial(
        pl.kernel,
        mesh=mesh,
        out_type=[jax.ShapeDtypeStruct((_NDEG,), jnp.float32),
                  jax.ShapeDtypeStruct((_NDEG,), jnp.float32)],
        compiler_params=pltpu.CompilerParams(use_tc_tiling_on_sc=False),
        scratch_types=[
            pltpu.VMEM((_CPW, _CH), jnp.int32),
            pltpu.VMEM((128,), jnp.float32),
            pltpu.VMEM((_DSEG,), jnp.float32),
            pltpu.VMEM_SHARED((_NDEG,), jnp.float32),
        ],
    )
    def deg_kernel(ei_hbm, out0_hbm, out1_hbm, didx, ones, zbuf, acc):
        c = lax.axis_index("c")
        s = lax.axis_index("s")
        w = c * 16 + s

        def fill(i, _):
            zbuf[pl.ds(i * 16, 16)] = jnp.zeros((16,), jnp.float32)
            return 0

        lax.fori_loop(0, _DSEG // 16, fill, 0)

        def fill1(i, _):
            ones[pl.ds(i * 16, 16)] = jnp.ones((16,), jnp.float32)
            return 0

        lax.fori_loop(0, 8, fill1, 0)
        pltpu.sync_copy(zbuf, acc.at[pl.ds(s * _DSEG, _DSEG)])
        pltpu.sync_copy(ei_hbm.at[1, w], didx)
        plsc.subcore_barrier()

        def body(j, _):
            pltpu.sync_copy(ones.at[pl.ds(0, _CH)], acc.at[didx.at[j]],
                            add=True)
            return 0

        lax.fori_loop(0, _CPW, body, 0)
        plsc.subcore_barrier()

        @pl.when(c == 0)
        def _():
            pltpu.sync_copy(acc.at[pl.ds(s * _DSEG, _DSEG)],
                            out0_hbm.at[pl.ds(s * _DSEG, _DSEG)])

        @pl.when(c == 1)
        def _():
            pltpu.sync_copy(acc.at[pl.ds(s * _DSEG, _DSEG)],
                            out1_hbm.at[pl.ds(s * _DSEG, _DSEG)])

    return deg_kernel


def _make_conv_kernel():
    mesh = plsc.VectorSubcoreMesh(core_axis_name="c", subcore_axis_name="s")

    @functools.partial(
        pl.kernel,
        mesh=mesh,
        out_type=jax.ShapeDtypeStruct((2, _NPAD, _G), jnp.float32),
        compiler_params=pltpu.CompilerParams(use_tc_tiling_on_sc=False),
        scratch_types=[
            pltpu.VMEM((_CPW, _CH), jnp.int32),
            pltpu.VMEM((_CPW, _CH), jnp.int32),
            pltpu.VMEM((_CH, _G), jnp.float32),
            pltpu.VMEM((_CH, _G), jnp.float32),
            pltpu.VMEM((_CH, _G), jnp.float32),
            pltpu.VMEM((_CH, _G), jnp.float32),
            pltpu.VMEM((_ZR, _G), jnp.float32),
            pltpu.VMEM_SHARED((_NPAD, _G), jnp.float32),
            pltpu.SemaphoreType.DMA,
            pltpu.SemaphoreType.DMA,
            pltpu.SemaphoreType.DMA,
            pltpu.SemaphoreType.DMA,
        ],
    )
    def conv_kernel(t_hbm, ei_hbm, out_hbm,
                    sidx, didx, r_a, r_b, r_c, r_d, zbuf, acc,
                    sem_a, sem_b, sem_c, sem_d):
        c = lax.axis_index("c")
        s = lax.axis_index("s")
        w = c * 16 + s

        pltpu.sync_copy(ei_hbm.at[0, w], sidx)
        pltpu.sync_copy(ei_hbm.at[1, w], didx)

        def gstart(j, buf, sem):
            pltpu.async_copy(t_hbm.at[sidx.at[j]], buf, sem)

        def gwait(j, buf, sem):
            pltpu.make_async_copy(t_hbm.at[sidx.at[j]], buf, sem).wait()

        gstart(0, r_a, sem_a)
        gstart(1, r_b, sem_b)
        gstart(2, r_c, sem_c)
        gstart(3, r_d, sem_d)

        def fill(i, _):
            zbuf[i, pl.ds(0, 16)] = jnp.zeros((16,), jnp.float32)
            zbuf[i, pl.ds(16, 16)] = jnp.zeros((16,), jnp.float32)
            zbuf[i, pl.ds(32, 16)] = jnp.zeros((16,), jnp.float32)
            zbuf[i, pl.ds(48, 16)] = jnp.zeros((16,), jnp.float32)
            return 0

        lax.fori_loop(0, _ZR, fill, 0)
        pltpu.sync_copy(zbuf, acc.at[pl.ds(s * _RPS, _ZR)])
        pltpu.sync_copy(zbuf, acc.at[pl.ds(s * _RPS + _ZR, _ZR)])
        plsc.subcore_barrier()

        def body(i, _):
            j0 = 4 * i

            def step(off, buf, sem):
                j = j0 + off
                gwait(j, buf, sem)
                pltpu.sync_copy(buf, acc.at[didx.at[j]], add=True)

                @pl.when(j + 4 < _CPW)
                def _():
                    gstart(j + 4, buf, sem)

            step(0, r_a, sem_a)
            step(1, r_b, sem_b)
            step(2, r_c, sem_c)
            step(3, r_d, sem_d)
            return 0

        lax.fori_loop(0, _CPW // 4, body, 0)
        plsc.subcore_barrier()
        pltpu.sync_copy(acc.at[pl.ds(s * _RPS, _RPS)],
                        out_hbm.at[c, pl.ds(s * _RPS, _RPS)])

    return conv_kernel


_deg_call = _make_deg_kernel()
_conv_call = _make_conv_kernel()


def _dot16(a, b):
    return jnp.dot(a.astype(jnp.bfloat16), b.astype(jnp.bfloat16),
                   preferred_element_type=jnp.float32)

def _tc0_body(x_ref, w_ref, d0_ref, d1_ref, t_ref, dinv_ref):
    deg = d0_ref[...] + d1_ref[...] + 1.0
    dinv = lax.rsqrt(deg)
    t = _dot16(x_ref[...], w_ref[...])
    t_ref[...] = t * dinv
    dinv_ref[...] = dinv


_tc0_call = pl.pallas_call(
    _tc0_body,
    grid=(_NB,),
    in_specs=[
        pl.BlockSpec((_RB, _D), lambda i: (i, 0)),
        pl.BlockSpec((_D, _G), lambda i: (0, 0)),
        pl.BlockSpec((_RB, 1), lambda i: (i, 0)),
        pl.BlockSpec((_RB, 1), lambda i: (i, 0)),
    ],
    out_specs=[
        pl.BlockSpec((_RB, _G), lambda i: (i, 0)),
        pl.BlockSpec((_RB, 1), lambda i: (i, 0)),
    ],
    out_shape=[
        jax.ShapeDtypeStruct((_N, _G), jnp.float32),
        jax.ShapeDtypeStruct((_N, 1), jnp.float32),
    ],
)


def _make_tc_conv(relu):
    def body(s_ref, t_ref, dinv_ref, b_ref, g_ref, bb_ref, w_ref,
             t_next_ref, stats_ref, u_scr):
        i = pl.program_id(0)
        j = lax.rem(i, _NB)

        @pl.when(i < _NB)
        def _():
            u = (s_ref[0] + s_ref[1] + t_ref[...]) * dinv_ref[...] + b_ref[...]
            if relu:
                u = jnp.maximum(u, 0.0)
            u_scr[pl.ds(j * _RB, _RB), :] = u

            @pl.when(i == 0)
            def _():
                stats_ref[2, :] = jnp.sum(u, axis=0) * (1.0 / _RB)

            a = stats_ref[2, :]
            d = u - a
            ps = jnp.sum(d, axis=0)
            pss = jnp.sum(d * d, axis=0)

            @pl.when(i == 0)
            def _():
                stats_ref[0, :] = ps
                stats_ref[1, :] = pss

            @pl.when(i > 0)
            def _():
                stats_ref[0, :] += ps
                stats_ref[1, :] += pss

        @pl.when(i >= _NB)
        def _():
            a = stats_ref[2, :]
            dm = stats_ref[0, :] * (1.0 / _N)
            mu = a + dm
            var = stats_ref[1, :] * (1.0 / _N) - dm * dm
            sc = lax.rsqrt(var + 1e-5) * g_ref[...]
            u = u_scr[pl.ds(j * _RB, _RB), :]
            h = (u - mu) * sc + bb_ref[...]
            t_next_ref[...] = _dot16(h, w_ref[...]) * dinv_ref[...]

    return pl.pallas_call(
        body,
        grid=(2 * _NB,),
        in_specs=[
            pl.BlockSpec((2, _RB, _G),
                         lambda i: (0, jnp.where(i < _NB, i, 0), 0)),
            pl.BlockSpec((_RB, _G), lambda i: (jnp.where(i < _NB, i, 0), 0)),
            pl.BlockSpec((_RB, 1), lambda i: (lax.rem(i, _NB), 0)),
            pl.BlockSpec((_G,), lambda i: (0,)),
            pl.BlockSpec((_G,), lambda i: (0,)),
            pl.BlockSpec((_G,), lambda i: (0,)),
            pl.BlockSpec((_G, _G), lambda i: (0, 0)),
        ],
        out_specs=pl.BlockSpec((_RB, _G), lambda i: (lax.max(i - _NB, 0), 0)),
        out_shape=jax.ShapeDtypeStruct((_N, _G), jnp.float32),
        scratch_shapes=[pltpu.VMEM((3, _G), jnp.float32),
                        pltpu.VMEM((_N, _G), jnp.float32)],
    )


_tc_conv_relu = _make_tc_conv(True)
_tc_conv_plain = _make_tc_conv(False)


def _bn_val(x, g, b):
    mu = jnp.mean(x, axis=0)
    d = x - mu
    var = jnp.mean(d * d, axis=0)
    return d * lax.rsqrt(var + 1e-5) * g + b


def _tc_tail_body(s_ref, t_ref, dinv_ref, b_ref, g_ref, bb_ref, batch_ref,
                  wm_ref, bm_ref, g2_ref, b2_ref, wd_ref, bd_ref,
                  gd_ref, bdn_ref, wo_ref, bo_ref,
                  out_ref, stats_ref, pooled_ref, u_scr):
    i = pl.program_id(0)
    j = lax.rem(i, _NB)

    @pl.when(i < _NB)
    def _():
        u = (s_ref[0] + s_ref[1] + t_ref[...]) * dinv_ref[...] + b_ref[...]
        u_scr[pl.ds(j * _RB, _RB), :] = u

        @pl.when(i == 0)
        def _():
            stats_ref[2, :] = jnp.sum(u, axis=0) * (1.0 / _RB)

        a = stats_ref[2, :]
        d = u - a
        ps = jnp.sum(d, axis=0)
        pss = jnp.sum(d * d, axis=0)

        @pl.when(i == 0)
        def _():
            stats_ref[0, :] = ps
            stats_ref[1, :] = pss

        @pl.when(i > 0)
        def _():
            stats_ref[0, :] += ps
            stats_ref[1, :] += pss

    @pl.when((i >= _NB) & (i < 2 * _NB))
    def _():
        a = stats_ref[2, :]
        dm = stats_ref[0, :] * (1.0 / _N)
        mu = a + dm
        var = stats_ref[1, :] * (1.0 / _N) - dm * dm
        sc = lax.rsqrt(var + 1e-5) * g_ref[...]
        u = u_scr[pl.ds(j * _RB, _RB), :]
        h = (u - mu) * sc + bb_ref[...]
        mask = (batch_ref[...] == lax.broadcasted_iota(jnp.int32, (1, _NG), 1)
                ).astype(jnp.float32)
        pp = lax.dot_general(mask, h, (((0,), (0,)), ((), ())),
                             preferred_element_type=jnp.float32,
                             precision=lax.Precision.HIGHEST)

        @pl.when(i == _NB)
        def _():
            pooled_ref[...] = pp

        @pl.when(i > _NB)
        def _():
            pooled_ref[...] += pp

    @pl.when(i == 2 * _NB)
    def _():
        p = pooled_ref[...]
        hm = _dot16(p, wm_ref[...])
        hm = jnp.maximum(hm + bm_ref[...], 0.0)
        hm = _bn_val(hm, g2_ref[...], b2_ref[...])
        for k in range(3):
            hm = _dot16(hm, wd_ref[k])
            hm = jnp.maximum(hm + bd_ref[k], 0.0)
            hm = _bn_val(hm, gd_ref[k], bdn_ref[k])
        out_ref[...] = _dot16(hm, wo_ref[...]) + bo_ref[...]


_tc_tail_call = pl.pallas_call(
    _tc_tail_body,
    grid=(2 * _NB + 1,),
    in_specs=[
        pl.BlockSpec((2, _RB, _G),
                     lambda i: (0, jnp.where(i < _NB, i, 0), 0)),
        pl.BlockSpec((_RB, _G), lambda i: (jnp.where(i < _NB, i, 0), 0)),
        pl.BlockSpec((_RB, 1), lambda i: (lax.rem(i, _NB), 0)),
        pl.BlockSpec((_G,), lambda i: (0,)),
        pl.BlockSpec((_G,), lambda i: (0,)),
        pl.BlockSpec((_G,), lambda i: (0,)),
        pl.BlockSpec((_RB, 1),
                     lambda i: (jnp.where((i >= _NB) & (i < 2 * _NB),
                                          i - _NB, 0), 0)),
        pl.BlockSpec((_G, _G), lambda i: (0, 0)),
        pl.BlockSpec((_G,), lambda i: (0,)),
        pl.BlockSpec((_G,), lambda i: (0,)),
        pl.BlockSpec((_G,), lambda i: (0,)),
        pl.BlockSpec((3, _G, _G), lambda i: (0, 0, 0)),
        pl.BlockSpec((3, _G), lambda i: (0, 0)),
        pl.BlockSpec((3, _G), lambda i: (0, 0)),
        pl.BlockSpec((3, _G), lambda i: (0, 0)),
        pl.BlockSpec((_G, 1), lambda i: (0, 0)),
        pl.BlockSpec((1,), lambda i: (0,)),
    ],
    out_specs=pl.BlockSpec((_NG, 1), lambda i: (0, 0)),
    out_shape=jax.ShapeDtypeStruct((_NG, 1), jnp.float32),
    scratch_shapes=[pltpu.VMEM((3, _G), jnp.float32),
                    pltpu.VMEM((_NG, _G), jnp.float32),
                    pltpu.VMEM((_N, _G), jnp.float32)],
)


def kernel(x, edge_index, batch, W1, b1, bn1_g, bn1_b, Wh, bh, bnh_g, bnh_b,
           Wm, bm, bn2_g, bn2_b, Wd, bd, bnd_g, bnd_b, Wo, bo):
    ei4d = edge_index.reshape(2, _NW, _CPW, _CH)
    batch2d = batch.reshape(_N, 1)

    deg0, deg1 = _deg_call(ei4d)
    t, dinv = _tc0_call(x, W1, deg0.reshape(_NDEG, 1), deg1.reshape(_NDEG, 1))

    biases = [b1, bh[0], bh[1], bh[2]]
    gammas = [bn1_g, bnh_g[0], bnh_g[1], bnh_g[2]]
    betas = [bn1_b, bnh_b[0], bnh_b[1], bnh_b[2]]
    nextw = [Wh[0], Wh[1], Wh[2]]

    for k in range(3):
        s_part = _conv_call(t, ei4d)
        tc = _tc_conv_relu if k == 0 else _tc_conv_plain
        t = tc(s_part, t, dinv, biases[k], gammas[k], betas[k], nextw[k])

    s_part = _conv_call(t, ei4d)
    return _tc_tail_call(s_part, t, dinv, biases[3], gammas[3], betas[3],
                         batch2d, Wm, bm, bn2_g, bn2_b, Wd, bd,
                         bnd_g, bnd_b, Wo, bo)


# speedup vs baseline: 1.2255x; 1.0013x over previous
import functools

import jax
import jax.numpy as jnp
from jax import lax
from jax.experimental import pallas as pl
from jax.experimental.pallas import tpu as pltpu
from jax.experimental.pallas import tpu_sc as plsc

_N = 10000
_E = 320000
_D = 128
_G = 64
_NG = 256
_CH = 125
_NW = 32
_EPW = _E // _NW
_CPW = _EPW // _CH
_RPS = 640
_NPAD = 16 * _RPS
_ZR = 320
_DSEG = 640
_NDEG = 16 * _DSEG
_RB = 1000
_NB = _N // _RB


def _make_deg_kernel():
    mesh = plsc.VectorSubcoreMesh(core_axis_name="c", subcore_axis_name="s")

    @functools.partial(
        pl.kernel,
        mesh=mesh,
        out_type=[jax.ShapeDtypeStruct((_NDEG,), jnp.float32),
                  jax.ShapeDtypeStruct((_NDEG,), jnp.float32)],
        compiler_params=pltpu.CompilerParams(use_tc_tiling_on_sc=False),
        scratch_types=[
            pltpu.VMEM((_CPW, _CH), jnp.int32),
            pltpu.VMEM((128,), jnp.float32),
            pltpu.VMEM((_DSEG,), jnp.float32),
            pltpu.VMEM_SHARED((_NDEG,), jnp.float32),
        ],
    )
    def deg_kernel(ei_hbm, out0_hbm, out1_hbm, didx, ones, zbuf, acc):
        c = lax.axis_index("c")
        s = lax.axis_index("s")
        w = c * 16 + s

        def fill(i, _):
            zbuf[pl.ds(i * 16, 16)] = jnp.zeros((16,), jnp.float32)
            return 0

        lax.fori_loop(0, _DSEG // 16, fill, 0)

        def fill1(i, _):
            ones[pl.ds(i * 16, 16)] = jnp.ones((16,), jnp.float32)
            return 0

        lax.fori_loop(0, 8, fill1, 0)
        pltpu.sync_copy(zbuf, acc.at[pl.ds(s * _DSEG, _DSEG)])
        pltpu.sync_copy(ei_hbm.at[1, w], didx)
        plsc.subcore_barrier()

        def body(j, _):
            pltpu.sync_copy(ones.at[pl.ds(0, _CH)], acc.at[didx.at[j]],
                            add=True)
            return 0

        lax.fori_loop(0, _CPW, body, 0)
        plsc.subcore_barrier()

        @pl.when(c == 0)
        def _():
            pltpu.sync_copy(acc.at[pl.ds(s * _DSEG, _DSEG)],
                            out0_hbm.at[pl.ds(s * _DSEG, _DSEG)])

        @pl.when(c == 1)
        def _():
            pltpu.sync_copy(acc.at[pl.ds(s * _DSEG, _DSEG)],
                            out1_hbm.at[pl.ds(s * _DSEG, _DSEG)])

    return deg_kernel


def _make_conv_kernel():
    mesh = plsc.VectorSubcoreMesh(core_axis_name="c", subcore_axis_name="s")

    @functools.partial(
        pl.kernel,
        mesh=mesh,
        out_type=jax.ShapeDtypeStruct((2, _NPAD, _G), jnp.float32),
        compiler_params=pltpu.CompilerParams(use_tc_tiling_on_sc=False),
        scratch_types=[
            pltpu.VMEM((_CPW, _CH), jnp.int32),
            pltpu.VMEM((_CPW, _CH), jnp.int32),
            pltpu.VMEM((_CH, _G), jnp.float32),
            pltpu.VMEM((_CH, _G), jnp.float32),
            pltpu.VMEM((_CH, _G), jnp.float32),
            pltpu.VMEM((_CH, _G), jnp.float32),
            pltpu.VMEM((_CH, _G), jnp.float32),
            pltpu.VMEM((_CH, _G), jnp.float32),
            pltpu.VMEM((_ZR, _G), jnp.float32),
            pltpu.VMEM_SHARED((_NPAD, _G), jnp.float32),
            pltpu.SemaphoreType.DMA,
            pltpu.SemaphoreType.DMA,
            pltpu.SemaphoreType.DMA,
            pltpu.SemaphoreType.DMA,
            pltpu.SemaphoreType.DMA,
            pltpu.SemaphoreType.DMA,
        ],
    )
    def conv_kernel(t_hbm, ei_hbm, out_hbm,
                    sidx, didx, r_a, r_b, r_c, r_d, r_e, r_f, zbuf, acc,
                    sem_a, sem_b, sem_c, sem_d, sem_e, sem_f):
        c = lax.axis_index("c")
        s = lax.axis_index("s")
        w = c * 16 + s

        pltpu.sync_copy(ei_hbm.at[0, w], sidx)
        pltpu.sync_copy(ei_hbm.at[1, w], didx)

        def gstart(j, buf, sem):
            pltpu.async_copy(t_hbm.at[sidx.at[j]], buf, sem)

        def gwait(j, buf, sem):
            pltpu.make_async_copy(t_hbm.at[sidx.at[j]], buf, sem).wait()

        gstart(0, r_a, sem_a)
        gstart(1, r_b, sem_b)
        gstart(2, r_c, sem_c)
        gstart(3, r_d, sem_d)
        gstart(4, r_e, sem_e)
        gstart(5, r_f, sem_f)

        def fill(i, _):
            zbuf[i, pl.ds(0, 16)] = jnp.zeros((16,), jnp.float32)
            zbuf[i, pl.ds(16, 16)] = jnp.zeros((16,), jnp.float32)
            zbuf[i, pl.ds(32, 16)] = jnp.zeros((16,), jnp.float32)
            zbuf[i, pl.ds(48, 16)] = jnp.zeros((16,), jnp.float32)
            return 0

        lax.fori_loop(0, _ZR, fill, 0)
        pltpu.sync_copy(zbuf, acc.at[pl.ds(s * _RPS, _ZR)])
        pltpu.sync_copy(zbuf, acc.at[pl.ds(s * _RPS + _ZR, _ZR)])
        plsc.subcore_barrier()

        def body(i, _):
            j0 = 6 * i

            def step(off, buf, sem):
                j = j0 + off
                gwait(j, buf, sem)
                pltpu.sync_copy(buf, acc.at[didx.at[j]], add=True)

                @pl.when(j + 6 < _CPW)
                def _():
                    gstart(j + 6, buf, sem)

            step(0, r_a, sem_a)
            step(1, r_b, sem_b)
            step(2, r_c, sem_c)
            step(3, r_d, sem_d)
            step(4, r_e, sem_e)
            step(5, r_f, sem_f)
            return 0

        lax.fori_loop(0, 13, body, 0)

        gwait(78, r_a, sem_a)
        pltpu.sync_copy(r_a, acc.at[didx.at[78]], add=True)
        gwait(79, r_b, sem_b)
        pltpu.sync_copy(r_b, acc.at[didx.at[79]], add=True)
        plsc.subcore_barrier()
        pltpu.sync_copy(acc.at[pl.ds(s * _RPS, _RPS)],
                        out_hbm.at[c, pl.ds(s * _RPS, _RPS)])

    return conv_kernel


_deg_call = _make_deg_kernel()
_conv_call = _make_conv_kernel()


def _dot16(a, b):
    return jnp.dot(a.astype(jnp.bfloat16), b.astype(jnp.bfloat16),
                   preferred_element_type=jnp.float32)

def _tc0_body(x_ref, w_ref, d0_ref, d1_ref, t_ref, dinv_ref):
    deg = d0_ref[...] + d1_ref[...] + 1.0
    dinv = lax.rsqrt(deg)
    t = _dot16(x_ref[...], w_ref[...])
    t_ref[...] = t * dinv
    dinv_ref[...] = dinv


_tc0_call = pl.pallas_call(
    _tc0_body,
    grid=(_NB,),
    in_specs=[
        pl.BlockSpec((_RB, _D), lambda i: (i, 0)),
        pl.BlockSpec((_D, _G), lambda i: (0, 0)),
        pl.BlockSpec((_RB, 1), lambda i: (i, 0)),
        pl.BlockSpec((_RB, 1), lambda i: (i, 0)),
    ],
    out_specs=[
        pl.BlockSpec((_RB, _G), lambda i: (i, 0)),
        pl.BlockSpec((_RB, 1), lambda i: (i, 0)),
    ],
    out_shape=[
        jax.ShapeDtypeStruct((_N, _G), jnp.float32),
        jax.ShapeDtypeStruct((_N, 1), jnp.float32),
    ],
)


def _make_tc_conv(relu):
    def body(s_ref, t_ref, dinv_ref, b_ref, g_ref, bb_ref, w_ref,
             t_next_ref, stats_ref, u_scr):
        i = pl.program_id(0)
        j = lax.rem(i, _NB)

        @pl.when(i < _NB)
        def _():
            u = (s_ref[0] + s_ref[1] + t_ref[...]) * dinv_ref[...] + b_ref[...]
            if relu:
                u = jnp.maximum(u, 0.0)
            u_scr[pl.ds(j * _RB, _RB), :] = u

            @pl.when(i == 0)
            def _():
                stats_ref[2, :] = jnp.sum(u, axis=0) * (1.0 / _RB)

            a = stats_ref[2, :]
            d = u - a
            ps = jnp.sum(d, axis=0)
            pss = jnp.sum(d * d, axis=0)

            @pl.when(i == 0)
            def _():
                stats_ref[0, :] = ps
                stats_ref[1, :] = pss

            @pl.when(i > 0)
            def _():
                stats_ref[0, :] += ps
                stats_ref[1, :] += pss

        @pl.when(i >= _NB)
        def _():
            a = stats_ref[2, :]
            dm = stats_ref[0, :] * (1.0 / _N)
            mu = a + dm
            var = stats_ref[1, :] * (1.0 / _N) - dm * dm
            sc = lax.rsqrt(var + 1e-5) * g_ref[...]
            u = u_scr[pl.ds(j * _RB, _RB), :]
            h = (u - mu) * sc + bb_ref[...]
            t_next_ref[...] = _dot16(h, w_ref[...]) * dinv_ref[...]

    return pl.pallas_call(
        body,
        grid=(2 * _NB,),
        in_specs=[
            pl.BlockSpec((2, _RB, _G),
                         lambda i: (0, jnp.where(i < _NB, i, 0), 0)),
            pl.BlockSpec((_RB, _G), lambda i: (jnp.where(i < _NB, i, 0), 0)),
            pl.BlockSpec((_RB, 1), lambda i: (lax.rem(i, _NB), 0)),
            pl.BlockSpec((_G,), lambda i: (0,)),
            pl.BlockSpec((_G,), lambda i: (0,)),
            pl.BlockSpec((_G,), lambda i: (0,)),
            pl.BlockSpec((_G, _G), lambda i: (0, 0)),
        ],
        out_specs=pl.BlockSpec((_RB, _G), lambda i: (lax.max(i - _NB, 0), 0)),
        out_shape=jax.ShapeDtypeStruct((_N, _G), jnp.float32),
        scratch_shapes=[pltpu.VMEM((3, _G), jnp.float32),
                        pltpu.VMEM((_N, _G), jnp.float32)],
    )


_tc_conv_relu = _make_tc_conv(True)
_tc_conv_plain = _make_tc_conv(False)


def _bn_val(x, g, b):
    mu = jnp.mean(x, axis=0)
    d = x - mu
    var = jnp.mean(d * d, axis=0)
    return d * lax.rsqrt(var + 1e-5) * g + b


def _tc_tail_body(s_ref, t_ref, dinv_ref, b_ref, g_ref, bb_ref, batch_ref,
                  wm_ref, bm_ref, g2_ref, b2_ref, wd_ref, bd_ref,
                  gd_ref, bdn_ref, wo_ref, bo_ref,
                  out_ref, stats_ref, pooled_ref, u_scr):
    i = pl.program_id(0)
    j = lax.rem(i, _NB)

    @pl.when(i < _NB)
    def _():
        u = (s_ref[0] + s_ref[1] + t_ref[...]) * dinv_ref[...] + b_ref[...]
        u_scr[pl.ds(j * _RB, _RB), :] = u

        @pl.when(i == 0)
        def _():
            stats_ref[2, :] = jnp.sum(u, axis=0) * (1.0 / _RB)

        a = stats_ref[2, :]
        d = u - a
        ps = jnp.sum(d, axis=0)
        pss = jnp.sum(d * d, axis=0)

        @pl.when(i == 0)
        def _():
            stats_ref[0, :] = ps
            stats_ref[1, :] = pss

        @pl.when(i > 0)
        def _():
            stats_ref[0, :] += ps
            stats_ref[1, :] += pss

    @pl.when((i >= _NB) & (i < 2 * _NB))
    def _():
        a = stats_ref[2, :]
        dm = stats_ref[0, :] * (1.0 / _N)
        mu = a + dm
        var = stats_ref[1, :] * (1.0 / _N) - dm * dm
        sc = lax.rsqrt(var + 1e-5) * g_ref[...]
        u = u_scr[pl.ds(j * _RB, _RB), :]
        h = (u - mu) * sc + bb_ref[...]
        mask = (batch_ref[...] == lax.broadcasted_iota(jnp.int32, (1, _NG), 1)
                ).astype(jnp.float32)
        pp = lax.dot_general(mask, h, (((0,), (0,)), ((), ())),
                             preferred_element_type=jnp.float32,
                             precision=lax.Precision.HIGHEST)

        @pl.when(i == _NB)
        def _():
            pooled_ref[...] = pp

        @pl.when(i > _NB)
        def _():
            pooled_ref[...] += pp

    @pl.when(i == 2 * _NB)
    def _():
        p = pooled_ref[...]
        hm = _dot16(p, wm_ref[...])
        hm = jnp.maximum(hm + bm_ref[...], 0.0)
        hm = _bn_val(hm, g2_ref[...], b2_ref[...])
        for k in range(3):
            hm = _dot16(hm, wd_ref[k])
            hm = jnp.maximum(hm + bd_ref[k], 0.0)
            hm = _bn_val(hm, gd_ref[k], bdn_ref[k])
        out_ref[...] = _dot16(hm, wo_ref[...]) + bo_ref[...]


_tc_tail_call = pl.pallas_call(
    _tc_tail_body,
    grid=(2 * _NB + 1,),
    in_specs=[
        pl.BlockSpec((2, _RB, _G),
                     lambda i: (0, jnp.where(i < _NB, i, 0), 0)),
        pl.BlockSpec((_RB, _G), lambda i: (jnp.where(i < _NB, i, 0), 0)),
        pl.BlockSpec((_RB, 1), lambda i: (lax.rem(i, _NB), 0)),
        pl.BlockSpec((_G,), lambda i: (0,)),
        pl.BlockSpec((_G,), lambda i: (0,)),
        pl.BlockSpec((_G,), lambda i: (0,)),
        pl.BlockSpec((_RB, 1),
                     lambda i: (jnp.where((i >= _NB) & (i < 2 * _NB),
                                          i - _NB, 0), 0)),
        pl.BlockSpec((_G, _G), lambda i: (0, 0)),
        pl.BlockSpec((_G,), lambda i: (0,)),
        pl.BlockSpec((_G,), lambda i: (0,)),
        pl.BlockSpec((_G,), lambda i: (0,)),
        pl.BlockSpec((3, _G, _G), lambda i: (0, 0, 0)),
        pl.BlockSpec((3, _G), lambda i: (0, 0)),
        pl.BlockSpec((3, _G), lambda i: (0, 0)),
        pl.BlockSpec((3, _G), lambda i: (0, 0)),
        pl.BlockSpec((_G, 1), lambda i: (0, 0)),
        pl.BlockSpec((1,), lambda i: (0,)),
    ],
    out_specs=pl.BlockSpec((_NG, 1), lambda i: (0, 0)),
    out_shape=jax.ShapeDtypeStruct((_NG, 1), jnp.float32),
    scratch_shapes=[pltpu.VMEM((3, _G), jnp.float32),
                    pltpu.VMEM((_NG, _G), jnp.float32),
                    pltpu.VMEM((_N, _G), jnp.float32)],
)


def kernel(x, edge_index, batch, W1, b1, bn1_g, bn1_b, Wh, bh, bnh_g, bnh_b,
           Wm, bm, bn2_g, bn2_b, Wd, bd, bnd_g, bnd_b, Wo, bo):
    ei4d = edge_index.reshape(2, _NW, _CPW, _CH)
    batch2d = batch.reshape(_N, 1)

    deg0, deg1 = _deg_call(ei4d)
    t, dinv = _tc0_call(x, W1, deg0.reshape(_NDEG, 1), deg1.reshape(_NDEG, 1))

    biases = [b1, bh[0], bh[1], bh[2]]
    gammas = [bn1_g, bnh_g[0], bnh_g[1], bnh_g[2]]
    betas = [bn1_b, bnh_b[0], bnh_b[1], bnh_b[2]]
    nextw = [Wh[0], Wh[1], Wh[2]]

    for k in range(3):
        s_part = _conv_call(t, ei4d)
        tc = _tc_conv_relu if k == 0 else _tc_conv_plain
        t = tc(s_part, t, dinv, biases[k], gammas[k], betas[k], nextw[k])

    s_part = _conv_call(t, ei4d)
    return _tc_tail_call(s_part, t, dinv, biases[3], gammas[3], betas[3],
                         batch2d, Wm, bm, bn2_g, bn2_b, Wd, bd,
                         bnd_g, bnd_b, Wo, bo)
